# baseline (speedup 1.0000x reference)
import functools

import jax
import jax.numpy as jnp
from jax import lax
from jax.experimental import pallas as pl
from jax.experimental.pallas import tpu as pltpu
from jax.experimental.pallas import tpu_sc as plsc

_N = 10000
_NP = 10240
_E = 320000
_K = 80
_NCH = _E // _K
_G = 64
_RPT = _NP // 16
_BLK = 1280
_GRID = _NP // _BLK
_NSLOT = 4
_EM = _E + _NP
_NCHM = _EM // _K
_GM = _G * _NP


@functools.cache
def _make_sc_degree():
    return functools.partial(
        pl.kernel,
        out_type=jax.ShapeDtypeStruct((2, _NP), jnp.float32),
        mesh=plsc.VectorSubcoreMesh(core_axis_name="c", subcore_axis_name="s"),
        scratch_types=[
            pltpu.VMEM((_K,), jnp.int32),
            pltpu.VMEM((_K,), jnp.float32),
            pltpu.VMEM((_RPT,), jnp.float32),
            pltpu.VMEM_SHARED((_NP,), jnp.float32),
        ],
    )(_sc_degree_body)


def _sc_degree_body(dst_hbm, out_hbm, idx_v, ones_v, zeros_v, acc):
    c = lax.axis_index("c")
    s = lax.axis_index("s")
    for k in range(_K // 16):
        ones_v[pl.ds(16 * k, 16)] = jnp.full((16,), 1.0, jnp.float32)

    def zbody(k, carry):
        zeros_v[pl.ds(16 * k, 16)] = jnp.zeros((16,), jnp.float32)
        return carry

    lax.fori_loop(0, _RPT // 16, zbody, 0)
    r0 = s * _RPT
    pltpu.sync_copy(zeros_v, acc.at[pl.ds(r0, _RPT)])
    plsc.subcore_barrier()
    nch = _NCH // 32
    base = (c * 16 + s) * nch

    def body(j, carry):
        pltpu.sync_copy(dst_hbm.at[base + j], idx_v)
        pltpu.sync_copy(ones_v, acc.at[idx_v], add=True)
        return carry

    lax.fori_loop(0, nch, body, 0)
    plsc.subcore_barrier()
    pltpu.sync_copy(acc.at[pl.ds(r0, _RPT)], out_hbm.at[c, pl.ds(r0, _RPT)])


def _agg_scratch():
    return ([pltpu.VMEM((_K,), jnp.int32)] * (2 * _NSLOT)
            + [pltpu.VMEM((_K, 128), jnp.float32)] * _NSLOT
            + [pltpu.VMEM_SHARED((_NP, 128), jnp.float32)]
            + [pltpu.SemaphoreType.DMA] * (4 * _NSLOT))


def _agg_run(s, base, nch, src_hbm, dst_hbm, g_ref, s_ref, scr):
    sib = scr[0:_NSLOT]
    dib = scr[_NSLOT:2 * _NSLOT]
    rb = scr[2 * _NSLOT:3 * _NSLOT]
    acc = scr[3 * _NSLOT]
    ise = scr[3 * _NSLOT + 1:3 * _NSLOT + 1 + _NSLOT]
    dse = scr[3 * _NSLOT + 1 + _NSLOT:3 * _NSLOT + 1 + 2 * _NSLOT]
    gse = scr[3 * _NSLOT + 1 + 2 * _NSLOT:3 * _NSLOT + 1 + 3 * _NSLOT]
    sse = scr[3 * _NSLOT + 1 + 3 * _NSLOT:3 * _NSLOT + 1 + 4 * _NSLOT]
    for b in range(2):
        pltpu.async_copy(src_hbm.at[base + b], sib[b], ise[b])
        pltpu.async_copy(dst_hbm.at[base + b], dib[b], dse[b])
    r0 = s * _RPT
    pltpu.sync_copy(g_ref.at[pl.ds(r0, _RPT)], acc.at[pl.ds(r0, _RPT)])
    plsc.subcore_barrier()
    pltpu.make_async_copy(src_hbm.at[base], sib[0], ise[0]).wait()
    pltpu.async_copy(g_ref.at[sib[0]], rb[0], gse[0])

    def step(j, b):
        si = (b + 2) % _NSLOT

        @pl.when((j >= 2) & (j + 2 < nch))
        def _():
            pltpu.make_async_copy(rb[si], acc.at[dib[si]], sse[si]).wait()

        @pl.when(j + 2 < nch)
        def _():
            pltpu.async_copy(src_hbm.at[base + j + 2], sib[si], ise[si])
            pltpu.async_copy(dst_hbm.at[base + j + 2], dib[si], dse[si])

        sg = (b + 1) % _NSLOT

        @pl.when(j + 1 < nch)
        def _():
            pltpu.make_async_copy(src_hbm.at[base], sib[sg], ise[sg]).wait()
            pltpu.async_copy(g_ref.at[sib[sg]], rb[sg], gse[sg])

        @pl.when(j < nch)
        def _():
            pltpu.make_async_copy(g_ref.at[sib[b]], rb[b], gse[b]).wait()
            pltpu.make_async_copy(dst_hbm.at[base], dib[b], dse[b]).wait()
            pltpu.async_copy(rb[b], acc.at[dib[b]], sse[b], add=True)

    def body(i, carry):
        for b in range(_NSLOT):
            step(_NSLOT * i + b, b)
        return carry

    lax.fori_loop(0, (nch + _NSLOT - 1) // _NSLOT, body, 0)
    for b in range(_NSLOT):
        pltpu.make_async_copy(rb[b], acc.at[dib[b]], sse[b]).wait()
    plsc.subcore_barrier()
    pltpu.sync_copy(acc.at[pl.ds(r0, _RPT)], s_ref.at[pl.ds(r0, _RPT)])


@functools.cache
def _make_sc_agg():
    return functools.partial(
        pl.kernel,
        out_type=(jax.ShapeDtypeStruct((_NP, 128), jnp.float32),
                  jax.ShapeDtypeStruct((_NP, 128), jnp.float32)),
        mesh=plsc.VectorSubcoreMesh(core_axis_name="c", subcore_axis_name="s"),
        scratch_types=_agg_scratch(),
    )(_sc_agg_body)


def _sc_agg_body(glo_hbm, ghi_hbm, src_hbm, dst_hbm, slo_hbm, shi_hbm, *scr):
    c = lax.axis_index("c")
    s = lax.axis_index("s")
    nch = _NCH // 16

    @pl.when(c == 0)
    def _():
        _agg_run(s, s * nch, nch, src_hbm, dst_hbm, glo_hbm, slo_hbm, scr)

    @pl.when(c == 1)
    def _():
        _agg_run(s, s * nch, nch, src_hbm, dst_hbm, ghi_hbm, shi_hbm, scr)


@functools.cache
def _make_sc_agg_es():
    return functools.partial(
        pl.kernel,
        out_type=(jax.ShapeDtypeStruct((_NP, 128), jnp.float32),
                  jax.ShapeDtypeStruct((_NP, 128), jnp.float32)),
        mesh=plsc.VectorSubcoreMesh(core_axis_name="c", subcore_axis_name="s"),
        scratch_types=_agg_scratch(),
    )(_sc_agg_es_body)


def _sc_agg_es_body(u_hbm, src_hbm, dst_hbm, p0_hbm, p1_hbm, *scr):
    c = lax.axis_index("c")
    s = lax.axis_index("s")
    nch = _NCH // 32

    @pl.when(c == 0)
    def _():
        _agg_run(s, s * nch, nch, src_hbm, dst_hbm, u_hbm, p0_hbm, scr)

    @pl.when(c == 1)
    def _():
        _agg_run(s, (16 + s) * nch, nch, src_hbm, dst_hbm, u_hbm, p1_hbm, scr)


@functools.cache
def _make_sc_pool():
    return functools.partial(
        pl.kernel,
        out_type=jax.ShapeDtypeStruct((2, _GM), jnp.float32),
        mesh=plsc.VectorSubcoreMesh(core_axis_name="c", subcore_axis_name="s"),
        scratch_types=[
            pltpu.VMEM((_K,), jnp.int32),
            pltpu.VMEM((_K,), jnp.int32),
            pltpu.VMEM((_K,), jnp.float32),
            pltpu.VMEM((_K,), jnp.int32),
            pltpu.VMEM((_K,), jnp.int32),
            pltpu.VMEM((_RPT,), jnp.float32),
            pltpu.VMEM_SHARED((_GM,), jnp.float32),
        ],
    )(_sc_pool_body)


def _sc_pool_body(srcm_hbm, dstm_hbm, wvec_hbm, gbase_hbm, out_hbm,
                  sib, dib, wv, gb, fl, zeros_v, acc):
    c = lax.axis_index("c")
    s = lax.axis_index("s")

    def zbody(k, carry):
        zeros_v[pl.ds(16 * k, 16)] = jnp.zeros((16,), jnp.float32)
        return carry

    lax.fori_loop(0, _RPT // 16, zbody, 0)
    r0 = s * (_GM // 16)

    def zcopy(k, carry):
        pltpu.sync_copy(zeros_v, acc.at[pl.ds(r0 + k * _RPT, _RPT)])
        return carry

    lax.fori_loop(0, _GM // 16 // _RPT, zcopy, 0)
    plsc.subcore_barrier()
    nch = _NCHM // 32
    base = (c * 16 + s) * nch

    def body(j, carry):
        pltpu.sync_copy(srcm_hbm.at[base + j], sib)
        pltpu.sync_copy(dstm_hbm.at[base + j], dib)
        pltpu.sync_copy(wvec_hbm.at[dib], wv)
        pltpu.sync_copy(gbase_hbm.at[dib], gb)
        for k in range(_K // 16):
            fl[pl.ds(16 * k, 16)] = (gb[pl.ds(16 * k, 16)]
                                     + sib[pl.ds(16 * k, 16)])
        pltpu.sync_copy(wv, acc.at[fl], add=True)
        return carry

    lax.fori_loop(0, nch, body, 0)
    plsc.subcore_barrier()
    pltpu.sync_copy(acc.at[pl.ds(r0, _GM // 16)],
                    out_hbm.at[c, pl.ds(r0, _GM // 16)])


def _scale_body(x_ref, dinv_ref, u_ref):
    u_ref[...] = x_ref[...] * dinv_ref[...]


_tc_scale = pl.pallas_call(
    _scale_body,
    grid=(_GRID,),
    in_specs=[
        pl.BlockSpec((_BLK, 128), lambda i: (i, 0)),
        pl.BlockSpec((_BLK, 1), lambda i: (i, 0)),
    ],
    out_specs=pl.BlockSpec((_BLK, 128), lambda i: (i, 0)),
    out_shape=jax.ShapeDtypeStruct((_NP, 128), jnp.float32),
)


def _first_body(p0_ref, p1_ref, u_ref, dinv_ref, b1_ref, w1_ref, w2_ref,
                glo_ref, ghi_ref):
    di = dinv_ref[...]
    t = (p0_ref[...] + p1_ref[...] - u_ref[...]) * di
    h1 = jnp.maximum(jnp.dot(t, w1_ref[...],
                             preferred_element_type=jnp.float32) + b1_ref[...],
                     0.0)
    g = jnp.dot(h1, w2_ref[...], preferred_element_type=jnp.float32) * di
    glo_ref[...] = g[:, :128]
    ghi_ref[...] = g[:, 128:]


_tc_first = pl.pallas_call(
    _first_body,
    grid=(_GRID,),
    in_specs=[
        pl.BlockSpec((_BLK, 128), lambda i: (i, 0)),
        pl.BlockSpec((_BLK, 128), lambda i: (i, 0)),
        pl.BlockSpec((_BLK, 128), lambda i: (i, 0)),
        pl.BlockSpec((_BLK, 1), lambda i: (i, 0)),
        pl.BlockSpec((1, 256), lambda i: (0, 0)),
        pl.BlockSpec((128, 256), lambda i: (0, 0)),
        pl.BlockSpec((256, 256), lambda i: (0, 0)),
    ],
    out_specs=[
        pl.BlockSpec((_BLK, 128), lambda i: (i, 0)),
        pl.BlockSpec((_BLK, 128), lambda i: (i, 0)),
    ],
    out_shape=[
        jax.ShapeDtypeStruct((_NP, 128), jnp.float32),
        jax.ShapeDtypeStruct((_NP, 128), jnp.float32),
    ],
)


def _layer_body(slo_ref, shi_ref, dinv_ref, b_ref, w_ref, glo_ref, ghi_ref):
    di = dinv_ref[...]
    h = jnp.concatenate([slo_ref[...], shi_ref[...]], axis=1)
    pre = jnp.maximum(h * di + b_ref[...], 0.0)
    g = jnp.dot(pre, w_ref[...], preferred_element_type=jnp.float32) * di
    glo_ref[...] = g[:, :128]
    ghi_ref[...] = g[:, 128:]


_tc_layer = pl.pallas_call(
    _layer_body,
    grid=(_GRID,),
    in_specs=[
        pl.BlockSpec((_BLK, 128), lambda i: (i, 0)),
        pl.BlockSpec((_BLK, 128), lambda i: (i, 0)),
        pl.BlockSpec((_BLK, 1), lambda i: (i, 0)),
        pl.BlockSpec((1, 256), lambda i: (0, 0)),
        pl.BlockSpec((256, 256), lambda i: (0, 0)),
    ],
    out_specs=[
        pl.BlockSpec((_BLK, 128), lambda i: (i, 0)),
        pl.BlockSpec((_BLK, 128), lambda i: (i, 0)),
    ],
    out_shape=[
        jax.ShapeDtypeStruct((_NP, 128), jnp.float32),
        jax.ShapeDtypeStruct((_NP, 128), jnp.float32),
    ],
)


def _final_body(m_ref, glo_ref, ghi_ref, b_ref, batch_ref, wl_ref, bl_ref,
                out_ref, psum, cnt):
    i = pl.program_id(0)

    @pl.when(i == 0)
    def _():
        psum[...] = jnp.zeros((_G, 256), jnp.float32)
        cnt[...] = jnp.zeros((_G, 1), jnp.float32)

    mb = m_ref[0] + m_ref[1]
    g5 = jnp.concatenate([glo_ref[...], ghi_ref[...]], axis=1)
    psum[...] += jnp.dot(mb, g5, preferred_element_type=jnp.float32)
    bt = batch_ref[0, 0, :]
    onehot = (bt[None, :] == lax.broadcasted_iota(jnp.int32, (_G, _BLK), 0)
              ).astype(jnp.float32)
    cnt[...] += jnp.sum(onehot, axis=1, keepdims=True)

    @pl.when(i == _GRID - 1)
    def _():
        pooled = (psum[...] + cnt[...] * b_ref[...]) / jnp.maximum(cnt[...],
                                                                   1.0)
        out_ref[...] = jnp.dot(pooled, wl_ref[...],
                               preferred_element_type=jnp.float32) + bl_ref[...]


_tc_final = pl.pallas_call(
    _final_body,
    grid=(_GRID,),
    in_specs=[
        pl.BlockSpec((2, _G, _BLK), lambda i: (0, 0, i)),
        pl.BlockSpec((_BLK, 128), lambda i: (i, 0)),
        pl.BlockSpec((_BLK, 128), lambda i: (i, 0)),
        pl.BlockSpec((1, 256), lambda i: (0, 0)),
        pl.BlockSpec((1, 1, _BLK), lambda i: (i, 0, 0)),
        pl.BlockSpec((256, 128), lambda i: (0, 0)),
        pl.BlockSpec((1, 128), lambda i: (0, 0)),
    ],
    out_specs=pl.BlockSpec((_G, 128), lambda i: (0, 0)),
    out_shape=jax.ShapeDtypeStruct((_G, 128), jnp.float32),
    scratch_shapes=[
        pltpu.VMEM((_G, 256), jnp.float32),
        pltpu.VMEM((_G, 1), jnp.float32),
    ],
)


def kernel(x, edge_index, batch, W1, b1, W2, b2, W3, b3, W4, b4, W5, b5,
           W_lin, b_lin):
    src2 = edge_index[0].reshape(_NCH, _K)
    dst2 = edge_index[1].reshape(_NCH, _K)
    xp = jnp.pad(x, ((0, _NP - _N), (0, 0)))
    batchp = jnp.pad(batch, (0, _NP - _N),
                     constant_values=_G).reshape(_GRID, 1, _BLK)
    loopn = jnp.arange(_NP, dtype=jnp.int32)
    srcm = jnp.concatenate([edge_index[0], loopn]).reshape(_NCHM, _K)
    dstm = jnp.concatenate([edge_index[1], loopn]).reshape(_NCHM, _K)

    degs = _make_sc_degree()(dst2)
    dinv = lax.rsqrt(degs[0] + degs[1] + 1.0).reshape(_NP, 1)
    real = loopn < _N
    wvec = jnp.where(real, dinv[:, 0], 0.0)
    gbase = jnp.where(real, jnp.pad(batch, (0, _NP - _N)), 0) * _NP

    agg128 = _make_sc_agg()

    u = _tc_scale(xp, dinv)
    p0, p1 = _make_sc_agg_es()(u, src2, dst2)
    glo, ghi = _tc_first(p0, p1, u, dinv, b1.reshape(1, 256), W1, W2)
    for b_prev, W in ((b2, W3), (b3, W4), (b4, W5)):
        slo, shi = agg128(glo, ghi, src2, dst2)
        glo, ghi = _tc_layer(slo, shi, dinv, b_prev.reshape(1, 256), W)
    m = _make_sc_pool()(srcm, dstm, wvec, gbase.astype(jnp.int32))
    return _tc_final(m.reshape(2, _G, _NP), glo, ghi, b5.reshape(1, 256),
                     batchp, W_lin, b_lin.reshape(1, 128))


# speedup vs baseline: 1.2865x; 1.2865x over previous
import functools

import jax
import jax.numpy as jnp
from jax import lax
from jax.experimental import pallas as pl
from jax.experimental.pallas import tpu as pltpu
from jax.experimental.pallas import tpu_sc as plsc

_N = 10000
_NP = 10240
_E = 320000
_K = 80
_NCH = _E // _K
_G = 64
_RPT = _NP // 16
_BLK = 1280
_GRID = _NP // _BLK
_NSLOT = 4
_EM = _E + _NP
_KM = 240
_NCHM = _EM // _KM
_GM = _G * _NP


@functools.cache
def _make_sc_degree():
    return functools.partial(
        pl.kernel,
        out_type=jax.ShapeDtypeStruct((2, _NP), jnp.float32),
        mesh=plsc.VectorSubcoreMesh(core_axis_name="c", subcore_axis_name="s"),
        scratch_types=[
            pltpu.VMEM((_K,), jnp.int32),
            pltpu.VMEM((_K,), jnp.float32),
            pltpu.VMEM((_RPT,), jnp.float32),
            pltpu.VMEM_SHARED((_NP,), jnp.float32),
        ],
    )(_sc_degree_body)


def _sc_degree_body(dst_hbm, out_hbm, idx_v, ones_v, zeros_v, acc):
    c = lax.axis_index("c")
    s = lax.axis_index("s")
    for k in range(_K // 16):
        ones_v[pl.ds(16 * k, 16)] = jnp.full((16,), 1.0, jnp.float32)

    def zbody(k, carry):
        zeros_v[pl.ds(16 * k, 16)] = jnp.zeros((16,), jnp.float32)
        return carry

    lax.fori_loop(0, _RPT // 16, zbody, 0)
    r0 = s * _RPT
    pltpu.sync_copy(zeros_v, acc.at[pl.ds(r0, _RPT)])
    plsc.subcore_barrier()
    nch = _NCH // 32
    base = (c * 16 + s) * nch

    def body(j, carry):
        pltpu.sync_copy(dst_hbm.at[base + j], idx_v)
        pltpu.sync_copy(ones_v, acc.at[idx_v], add=True)
        return carry

    lax.fori_loop(0, nch, body, 0)
    plsc.subcore_barrier()
    pltpu.sync_copy(acc.at[pl.ds(r0, _RPT)], out_hbm.at[c, pl.ds(r0, _RPT)])


def _agg_scratch():
    return ([pltpu.VMEM((_K,), jnp.int32)] * (2 * _NSLOT)
            + [pltpu.VMEM((_K, 128), jnp.float32)] * _NSLOT
            + [pltpu.VMEM_SHARED((_NP, 128), jnp.float32)]
            + [pltpu.SemaphoreType.DMA] * (4 * _NSLOT))


def _agg_run(s, base, nch, src_hbm, dst_hbm, g_ref, s_ref, scr):
    sib = scr[0:_NSLOT]
    dib = scr[_NSLOT:2 * _NSLOT]
    rb = scr[2 * _NSLOT:3 * _NSLOT]
    acc = scr[3 * _NSLOT]
    ise = scr[3 * _NSLOT + 1:3 * _NSLOT + 1 + _NSLOT]
    dse = scr[3 * _NSLOT + 1 + _NSLOT:3 * _NSLOT + 1 + 2 * _NSLOT]
    gse = scr[3 * _NSLOT + 1 + 2 * _NSLOT:3 * _NSLOT + 1 + 3 * _NSLOT]
    sse = scr[3 * _NSLOT + 1 + 3 * _NSLOT:3 * _NSLOT + 1 + 4 * _NSLOT]
    for b in range(2):
        pltpu.async_copy(src_hbm.at[base + b], sib[b], ise[b])
        pltpu.async_copy(dst_hbm.at[base + b], dib[b], dse[b])
    r0 = s * _RPT
    pltpu.sync_copy(g_ref.at[pl.ds(r0, _RPT)], acc.at[pl.ds(r0, _RPT)])
    plsc.subcore_barrier()
    pltpu.make_async_copy(src_hbm.at[base], sib[0], ise[0]).wait()
    pltpu.async_copy(g_ref.at[sib[0]], rb[0], gse[0])

    def step(j, b):
        si = (b + 2) % _NSLOT

        @pl.when((j >= 2) & (j + 2 < nch))
        def _():
            pltpu.make_async_copy(rb[si], acc.at[dib[si]], sse[si]).wait()

        @pl.when(j + 2 < nch)
        def _():
            pltpu.async_copy(src_hbm.at[base + j + 2], sib[si], ise[si])
            pltpu.async_copy(dst_hbm.at[base + j + 2], dib[si], dse[si])

        sg = (b + 1) % _NSLOT

        @pl.when(j + 1 < nch)
        def _():
            pltpu.make_async_copy(src_hbm.at[base], sib[sg], ise[sg]).wait()
            pltpu.async_copy(g_ref.at[sib[sg]], rb[sg], gse[sg])

        @pl.when(j < nch)
        def _():
            pltpu.make_async_copy(g_ref.at[sib[b]], rb[b], gse[b]).wait()
            pltpu.make_async_copy(dst_hbm.at[base], dib[b], dse[b]).wait()
            pltpu.async_copy(rb[b], acc.at[dib[b]], sse[b], add=True)

    def body(i, carry):
        for b in range(_NSLOT):
            step(_NSLOT * i + b, b)
        return carry

    lax.fori_loop(0, (nch + _NSLOT - 1) // _NSLOT, body, 0)
    for b in range(_NSLOT):
        pltpu.make_async_copy(rb[b], acc.at[dib[b]], sse[b]).wait()
    plsc.subcore_barrier()
    pltpu.sync_copy(acc.at[pl.ds(r0, _RPT)], s_ref.at[pl.ds(r0, _RPT)])


@functools.cache
def _make_sc_agg():
    return functools.partial(
        pl.kernel,
        out_type=(jax.ShapeDtypeStruct((_NP, 128), jnp.float32),
                  jax.ShapeDtypeStruct((_NP, 128), jnp.float32)),
        mesh=plsc.VectorSubcoreMesh(core_axis_name="c", subcore_axis_name="s"),
        scratch_types=_agg_scratch(),
    )(_sc_agg_body)


def _sc_agg_body(glo_hbm, ghi_hbm, src_hbm, dst_hbm, slo_hbm, shi_hbm, *scr):
    c = lax.axis_index("c")
    s = lax.axis_index("s")
    nch = _NCH // 16

    @pl.when(c == 0)
    def _():
        _agg_run(s, s * nch, nch, src_hbm, dst_hbm, glo_hbm, slo_hbm, scr)

    @pl.when(c == 1)
    def _():
        _agg_run(s, s * nch, nch, src_hbm, dst_hbm, ghi_hbm, shi_hbm, scr)


@functools.cache
def _make_sc_agg_es():
    return functools.partial(
        pl.kernel,
        out_type=(jax.ShapeDtypeStruct((_NP, 128), jnp.float32),
                  jax.ShapeDtypeStruct((_NP, 128), jnp.float32)),
        mesh=plsc.VectorSubcoreMesh(core_axis_name="c", subcore_axis_name="s"),
        scratch_types=_agg_scratch(),
    )(_sc_agg_es_body)


def _sc_agg_es_body(u_hbm, src_hbm, dst_hbm, p0_hbm, p1_hbm, *scr):
    c = lax.axis_index("c")
    s = lax.axis_index("s")
    nch = _NCH // 32

    @pl.when(c == 0)
    def _():
        _agg_run(s, s * nch, nch, src_hbm, dst_hbm, u_hbm, p0_hbm, scr)

    @pl.when(c == 1)
    def _():
        _agg_run(s, (16 + s) * nch, nch, src_hbm, dst_hbm, u_hbm, p1_hbm, scr)


@functools.cache
def _make_sc_pool():
    return functools.partial(
        pl.kernel,
        out_type=jax.ShapeDtypeStruct((2, _GM), jnp.float32),
        mesh=plsc.VectorSubcoreMesh(core_axis_name="c", subcore_axis_name="s"),
        scratch_types=(
            [pltpu.VMEM((_KM,), jnp.int32)] * 8
            + [pltpu.VMEM((_KM,), jnp.float32)] * 4
            + [pltpu.VMEM((_KM,), jnp.int32)] * 4
            + [pltpu.VMEM((_KM,), jnp.int32)] * 4
            + [pltpu.VMEM((2560,), jnp.float32)]
            + [pltpu.VMEM_SHARED((_GM,), jnp.float32)]
            + [pltpu.SemaphoreType.DMA] * 20
        ),
    )(_sc_pool_body)


def _sc_pool_body(srcm_hbm, dstm_hbm, wvec_hbm, gbase_hbm, out_hbm, *scr):
    c = lax.axis_index("c")
    s = lax.axis_index("s")
    sib = scr[0:4]
    dib = scr[4:8]
    wv = scr[8:12]
    gb = scr[12:16]
    fl = scr[16:20]
    zeros_v = scr[20]
    acc = scr[21]
    ise = scr[22:26]
    dse = scr[26:30]
    wse = scr[30:34]
    gse = scr[34:38]
    sse = scr[38:42]

    def zbody(k, carry):
        zeros_v[pl.ds(16 * k, 16)] = jnp.zeros((16,), jnp.float32)
        return carry

    lax.fori_loop(0, 2560 // 16, zbody, 0)
    r0 = s * (_GM // 16)

    def zcopy(k, carry):
        pltpu.sync_copy(zeros_v, acc.at[pl.ds(r0 + k * 2560, 2560)])
        return carry

    lax.fori_loop(0, _GM // 16 // 2560, zcopy, 0)
    plsc.subcore_barrier()
    nch = _NCHM // 32
    base = (c * 16 + s) * nch

    for b in range(2):
        pltpu.async_copy(srcm_hbm.at[base + b], sib[b], ise[b])
        pltpu.async_copy(dstm_hbm.at[base + b], dib[b], dse[b])
    pltpu.make_async_copy(dstm_hbm.at[base], dib[0], dse[0]).wait()
    pltpu.async_copy(wvec_hbm.at[dib[0]], wv[0], wse[0])
    pltpu.async_copy(gbase_hbm.at[dib[0]], gb[0], gse[0])

    def step(j, b):
        si = (b + 2) % 4

        @pl.when((j >= 2) & (j + 2 < nch))
        def _():
            pltpu.make_async_copy(wv[si], acc.at[fl[si]], sse[si]).wait()

        @pl.when(j + 2 < nch)
        def _():
            pltpu.async_copy(srcm_hbm.at[base + j + 2], sib[si], ise[si])
            pltpu.async_copy(dstm_hbm.at[base + j + 2], dib[si], dse[si])

        sg = (b + 1) % 4

        @pl.when(j + 1 < nch)
        def _():
            pltpu.make_async_copy(dstm_hbm.at[base], dib[sg], dse[sg]).wait()
            pltpu.async_copy(wvec_hbm.at[dib[sg]], wv[sg], wse[sg])
            pltpu.async_copy(gbase_hbm.at[dib[sg]], gb[sg], gse[sg])

        @pl.when(j < nch)
        def _():
            pltpu.make_async_copy(srcm_hbm.at[base], sib[b], ise[b]).wait()
            pltpu.make_async_copy(wvec_hbm.at[dib[b]], wv[b], wse[b]).wait()
            pltpu.make_async_copy(gbase_hbm.at[dib[b]], gb[b], gse[b]).wait()
            for k in range(_KM // 16):
                fl[b][pl.ds(16 * k, 16)] = (gb[b][pl.ds(16 * k, 16)]
                                            + sib[b][pl.ds(16 * k, 16)])
            pltpu.async_copy(wv[b], acc.at[fl[b]], sse[b], add=True)

    def body(i, carry):
        for b in range(4):
            step(4 * i + b, b)
        return carry

    lax.fori_loop(0, (nch + 3) // 4, body, 0)
    for b in range(4):
        pltpu.make_async_copy(wv[b], acc.at[fl[b]], sse[b]).wait()
    plsc.subcore_barrier()
    pltpu.sync_copy(acc.at[pl.ds(r0, _GM // 16)],
                    out_hbm.at[c, pl.ds(r0, _GM // 16)])


def _scale_body(x_ref, dinv_ref, u_ref):
    u_ref[...] = x_ref[...] * dinv_ref[...]


_tc_scale = pl.pallas_call(
    _scale_body,
    grid=(_GRID,),
    in_specs=[
        pl.BlockSpec((_BLK, 128), lambda i: (i, 0)),
        pl.BlockSpec((_BLK, 1), lambda i: (i, 0)),
    ],
    out_specs=pl.BlockSpec((_BLK, 128), lambda i: (i, 0)),
    out_shape=jax.ShapeDtypeStruct((_NP, 128), jnp.float32),
)


def _first_body(p0_ref, p1_ref, u_ref, dinv_ref, b1_ref, w1_ref, w2_ref,
                glo_ref, ghi_ref):
    di = dinv_ref[...]
    t = (p0_ref[...] + p1_ref[...] - u_ref[...]) * di
    h1 = jnp.maximum(jnp.dot(t, w1_ref[...],
                             preferred_element_type=jnp.float32) + b1_ref[...],
                     0.0)
    g = jnp.dot(h1, w2_ref[...], preferred_element_type=jnp.float32) * di
    glo_ref[...] = g[:, :128]
    ghi_ref[...] = g[:, 128:]


_tc_first = pl.pallas_call(
    _first_body,
    grid=(_GRID,),
    in_specs=[
        pl.BlockSpec((_BLK, 128), lambda i: (i, 0)),
        pl.BlockSpec((_BLK, 128), lambda i: (i, 0)),
        pl.BlockSpec((_BLK, 128), lambda i: (i, 0)),
        pl.BlockSpec((_BLK, 1), lambda i: (i, 0)),
        pl.BlockSpec((1, 256), lambda i: (0, 0)),
        pl.BlockSpec((128, 256), lambda i: (0, 0)),
        pl.BlockSpec((256, 256), lambda i: (0, 0)),
    ],
    out_specs=[
        pl.BlockSpec((_BLK, 128), lambda i: (i, 0)),
        pl.BlockSpec((_BLK, 128), lambda i: (i, 0)),
    ],
    out_shape=[
        jax.ShapeDtypeStruct((_NP, 128), jnp.float32),
        jax.ShapeDtypeStruct((_NP, 128), jnp.float32),
    ],
)


def _layer_body(slo_ref, shi_ref, dinv_ref, b_ref, w_ref, glo_ref, ghi_ref):
    di = dinv_ref[...]
    h = jnp.concatenate([slo_ref[...], shi_ref[...]], axis=1)
    pre = jnp.maximum(h * di + b_ref[...], 0.0)
    g = jnp.dot(pre, w_ref[...], preferred_element_type=jnp.float32) * di
    glo_ref[...] = g[:, :128]
    ghi_ref[...] = g[:, 128:]


_tc_layer = pl.pallas_call(
    _layer_body,
    grid=(_GRID,),
    in_specs=[
        pl.BlockSpec((_BLK, 128), lambda i: (i, 0)),
        pl.BlockSpec((_BLK, 128), lambda i: (i, 0)),
        pl.BlockSpec((_BLK, 1), lambda i: (i, 0)),
        pl.BlockSpec((1, 256), lambda i: (0, 0)),
        pl.BlockSpec((256, 256), lambda i: (0, 0)),
    ],
    out_specs=[
        pl.BlockSpec((_BLK, 128), lambda i: (i, 0)),
        pl.BlockSpec((_BLK, 128), lambda i: (i, 0)),
    ],
    out_shape=[
        jax.ShapeDtypeStruct((_NP, 128), jnp.float32),
        jax.ShapeDtypeStruct((_NP, 128), jnp.float32),
    ],
)


def _final_body(m_ref, glo_ref, ghi_ref, b_ref, batch_ref, wl_ref, bl_ref,
                out_ref, psum, cnt):
    i = pl.program_id(0)

    @pl.when(i == 0)
    def _():
        psum[...] = jnp.zeros((_G, 256), jnp.float32)
        cnt[...] = jnp.zeros((_G, 1), jnp.float32)

    mb = m_ref[0] + m_ref[1]
    g5 = jnp.concatenate([glo_ref[...], ghi_ref[...]], axis=1)
    psum[...] += jnp.dot(mb, g5, preferred_element_type=jnp.float32)
    bt = batch_ref[0, 0, :]
    onehot = (bt[None, :] == lax.broadcasted_iota(jnp.int32, (_G, _BLK), 0)
              ).astype(jnp.float32)
    cnt[...] += jnp.sum(onehot, axis=1, keepdims=True)

    @pl.when(i == _GRID - 1)
    def _():
        pooled = (psum[...] + cnt[...] * b_ref[...]) / jnp.maximum(cnt[...],
                                                                   1.0)
        out_ref[...] = jnp.dot(pooled, wl_ref[...],
                               preferred_element_type=jnp.float32) + bl_ref[...]


_tc_final = pl.pallas_call(
    _final_body,
    grid=(_GRID,),
    in_specs=[
        pl.BlockSpec((2, _G, _BLK), lambda i: (0, 0, i)),
        pl.BlockSpec((_BLK, 128), lambda i: (i, 0)),
        pl.BlockSpec((_BLK, 128), lambda i: (i, 0)),
        pl.BlockSpec((1, 256), lambda i: (0, 0)),
        pl.BlockSpec((1, 1, _BLK), lambda i: (i, 0, 0)),
        pl.BlockSpec((256, 128), lambda i: (0, 0)),
        pl.BlockSpec((1, 128), lambda i: (0, 0)),
    ],
    out_specs=pl.BlockSpec((_G, 128), lambda i: (0, 0)),
    out_shape=jax.ShapeDtypeStruct((_G, 128), jnp.float32),
    scratch_shapes=[
        pltpu.VMEM((_G, 256), jnp.float32),
        pltpu.VMEM((_G, 1), jnp.float32),
    ],
)


def kernel(x, edge_index, batch, W1, b1, W2, b2, W3, b3, W4, b4, W5, b5,
           W_lin, b_lin):
    src2 = edge_index[0].reshape(_NCH, _K)
    dst2 = edge_index[1].reshape(_NCH, _K)
    xp = jnp.pad(x, ((0, _NP - _N), (0, 0)))
    batchp = jnp.pad(batch, (0, _NP - _N),
                     constant_values=_G).reshape(_GRID, 1, _BLK)
    loopn = jnp.arange(_NP, dtype=jnp.int32)
    srcm = jnp.concatenate([edge_index[0], loopn]).reshape(_NCHM, _KM)
    dstm = jnp.concatenate([edge_index[1], loopn]).reshape(_NCHM, _KM)

    degs = _make_sc_degree()(dst2)
    dinv = lax.rsqrt(degs[0] + degs[1] + 1.0).reshape(_NP, 1)
    real = loopn < _N
    wvec = jnp.where(real, dinv[:, 0], 0.0)
    gbase = jnp.where(real, jnp.pad(batch, (0, _NP - _N)), 0) * _NP

    agg128 = _make_sc_agg()

    u = _tc_scale(xp, dinv)
    p0, p1 = _make_sc_agg_es()(u, src2, dst2)
    glo, ghi = _tc_first(p0, p1, u, dinv, b1.reshape(1, 256), W1, W2)
    for b_prev, W in ((b2, W3), (b3, W4), (b4, W5)):
        slo, shi = agg128(glo, ghi, src2, dst2)
        glo, ghi = _tc_layer(slo, shi, dinv, b_prev.reshape(1, 256), W)
    m = _make_sc_pool()(srcm, dstm, wvec, gbase.astype(jnp.int32))
    return _tc_final(m.reshape(2, _G, _NP), glo, ghi, b5.reshape(1, 256),
                     batchp, W_lin, b_lin.reshape(1, 128))


# speedup vs baseline: 1.4013x; 1.0892x over previous
import functools

import jax
import jax.numpy as jnp
from jax import lax
from jax.experimental import pallas as pl
from jax.experimental.pallas import tpu as pltpu
from jax.experimental.pallas import tpu_sc as plsc

_N = 10000
_NP = 10240
_E = 320000
_K = 80
_NCH = _E // _K
_G = 64
_RPT = _NP // 16
_BLK = 1280
_GRID = _NP // _BLK
_NSLOT = 4
_EM = _E + _NP
_KM = 240
_NCHM = _EM // _KM
_GM = _G * _NP


_KD = 400
_NCHD = _E // _KD


@functools.cache
def _make_sc_degree():
    return functools.partial(
        pl.kernel,
        out_type=jax.ShapeDtypeStruct((2, _NP), jnp.float32),
        mesh=plsc.VectorSubcoreMesh(core_axis_name="c", subcore_axis_name="s"),
        scratch_types=(
            [pltpu.VMEM((_KD,), jnp.int32)] * 4
            + [pltpu.VMEM((_KD,), jnp.float32)]
            + [pltpu.VMEM((_RPT,), jnp.float32)]
            + [pltpu.VMEM_SHARED((_NP,), jnp.float32)]
            + [pltpu.SemaphoreType.DMA] * 8
        ),
    )(_sc_degree_body)


def _sc_degree_body(dst_hbm, out_hbm, *scr):
    c = lax.axis_index("c")
    s = lax.axis_index("s")
    dib = scr[0:4]
    ones_v = scr[4]
    zeros_v = scr[5]
    acc = scr[6]
    dse = scr[7:11]
    sse = scr[11:15]
    nch = _NCHD // 32
    base = (c * 16 + s) * nch
    for b in range(2):
        pltpu.async_copy(dst_hbm.at[base + b], dib[b], dse[b])
    for k in range(_KD // 16):
        ones_v[pl.ds(16 * k, 16)] = jnp.full((16,), 1.0, jnp.float32)

    def zbody(k, carry):
        zeros_v[pl.ds(16 * k, 16)] = jnp.zeros((16,), jnp.float32)
        return carry

    lax.fori_loop(0, _RPT // 16, zbody, 0)
    r0 = s * _RPT
    pltpu.sync_copy(zeros_v, acc.at[pl.ds(r0, _RPT)])
    plsc.subcore_barrier()

    def step(j, b):
        si = (b + 2) % 4

        @pl.when((j >= 2) & (j + 2 < nch))
        def _():
            pltpu.make_async_copy(ones_v, acc.at[dib[si]], sse[si]).wait()

        @pl.when(j + 2 < nch)
        def _():
            pltpu.async_copy(dst_hbm.at[base + j + 2], dib[si], dse[si])

        @pl.when(j < nch)
        def _():
            pltpu.make_async_copy(dst_hbm.at[base], dib[b], dse[b]).wait()
            pltpu.async_copy(ones_v, acc.at[dib[b]], sse[b], add=True)

    def body(i, carry):
        for b in range(4):
            step(4 * i + b, b)
        return carry

    lax.fori_loop(0, (nch + 3) // 4, body, 0)
    for b in range(4):
        pltpu.make_async_copy(ones_v, acc.at[dib[b]], sse[b]).wait()
    plsc.subcore_barrier()
    pltpu.sync_copy(acc.at[pl.ds(r0, _RPT)], out_hbm.at[c, pl.ds(r0, _RPT)])


def _agg_scratch():
    return ([pltpu.VMEM((_K,), jnp.int32)] * (2 * _NSLOT)
            + [pltpu.VMEM((_K, 128), jnp.float32)] * _NSLOT
            + [pltpu.VMEM_SHARED((_NP, 128), jnp.float32)]
            + [pltpu.SemaphoreType.DMA] * (4 * _NSLOT))


def _agg_run(s, base, nch, src_hbm, dst_hbm, g_ref, s_ref, scr):
    sib = scr[0:_NSLOT]
    dib = scr[_NSLOT:2 * _NSLOT]
    rb = scr[2 * _NSLOT:3 * _NSLOT]
    acc = scr[3 * _NSLOT]
    ise = scr[3 * _NSLOT + 1:3 * _NSLOT + 1 + _NSLOT]
    dse = scr[3 * _NSLOT + 1 + _NSLOT:3 * _NSLOT + 1 + 2 * _NSLOT]
    gse = scr[3 * _NSLOT + 1 + 2 * _NSLOT:3 * _NSLOT + 1 + 3 * _NSLOT]
    sse = scr[3 * _NSLOT + 1 + 3 * _NSLOT:3 * _NSLOT + 1 + 4 * _NSLOT]
    for b in range(2):
        pltpu.async_copy(src_hbm.at[base + b], sib[b], ise[b])
        pltpu.async_copy(dst_hbm.at[base + b], dib[b], dse[b])
    r0 = s * _RPT
    pltpu.sync_copy(g_ref.at[pl.ds(r0, _RPT)], acc.at[pl.ds(r0, _RPT)])
    plsc.subcore_barrier()
    pltpu.make_async_copy(src_hbm.at[base], sib[0], ise[0]).wait()
    pltpu.async_copy(g_ref.at[sib[0]], rb[0], gse[0])

    def step(j, b):
        si = (b + 2) % _NSLOT

        @pl.when((j >= 2) & (j + 2 < nch))
        def _():
            pltpu.make_async_copy(rb[si], acc.at[dib[si]], sse[si]).wait()

        @pl.when(j + 2 < nch)
        def _():
            pltpu.async_copy(src_hbm.at[base + j + 2], sib[si], ise[si])
            pltpu.async_copy(dst_hbm.at[base + j + 2], dib[si], dse[si])

        sg = (b + 1) % _NSLOT

        @pl.when(j + 1 < nch)
        def _():
            pltpu.make_async_copy(src_hbm.at[base], sib[sg], ise[sg]).wait()
            pltpu.async_copy(g_ref.at[sib[sg]], rb[sg], gse[sg])

        @pl.when(j < nch)
        def _():
            pltpu.make_async_copy(g_ref.at[sib[b]], rb[b], gse[b]).wait()
            pltpu.make_async_copy(dst_hbm.at[base], dib[b], dse[b]).wait()
            pltpu.async_copy(rb[b], acc.at[dib[b]], sse[b], add=True)

    def body(i, carry):
        for b in range(_NSLOT):
            step(_NSLOT * i + b, b)
        return carry

    lax.fori_loop(0, (nch + _NSLOT - 1) // _NSLOT, body, 0)
    for b in range(_NSLOT):
        pltpu.make_async_copy(rb[b], acc.at[dib[b]], sse[b]).wait()
    plsc.subcore_barrier()
    pltpu.sync_copy(acc.at[pl.ds(r0, _RPT)], s_ref.at[pl.ds(r0, _RPT)])


@functools.cache
def _make_sc_agg():
    return functools.partial(
        pl.kernel,
        out_type=(jax.ShapeDtypeStruct((_NP, 128), jnp.float32),
                  jax.ShapeDtypeStruct((_NP, 128), jnp.float32)),
        mesh=plsc.VectorSubcoreMesh(core_axis_name="c", subcore_axis_name="s"),
        scratch_types=_agg_scratch(),
    )(_sc_agg_body)


def _sc_agg_body(glo_hbm, ghi_hbm, src_hbm, dst_hbm, slo_hbm, shi_hbm, *scr):
    c = lax.axis_index("c")
    s = lax.axis_index("s")
    nch = _NCH // 16

    @pl.when(c == 0)
    def _():
        _agg_run(s, s * nch, nch, src_hbm, dst_hbm, glo_hbm, slo_hbm, scr)

    @pl.when(c == 1)
    def _():
        _agg_run(s, s * nch, nch, src_hbm, dst_hbm, ghi_hbm, shi_hbm, scr)


@functools.cache
def _make_sc_agg_es():
    return functools.partial(
        pl.kernel,
        out_type=(jax.ShapeDtypeStruct((_NP, 128), jnp.float32),
                  jax.ShapeDtypeStruct((_NP, 128), jnp.float32)),
        mesh=plsc.VectorSubcoreMesh(core_axis_name="c", subcore_axis_name="s"),
        scratch_types=_agg_scratch(),
    )(_sc_agg_es_body)


def _sc_agg_es_body(u_hbm, src_hbm, dst_hbm, p0_hbm, p1_hbm, *scr):
    c = lax.axis_index("c")
    s = lax.axis_index("s")
    nch = _NCH // 32

    @pl.when(c == 0)
    def _():
        _agg_run(s, s * nch, nch, src_hbm, dst_hbm, u_hbm, p0_hbm, scr)

    @pl.when(c == 1)
    def _():
        _agg_run(s, (16 + s) * nch, nch, src_hbm, dst_hbm, u_hbm, p1_hbm, scr)


@functools.cache
def _make_sc_pool():
    return functools.partial(
        pl.kernel,
        out_type=jax.ShapeDtypeStruct((2, _GM), jnp.float32),
        mesh=plsc.VectorSubcoreMesh(core_axis_name="c", subcore_axis_name="s"),
        scratch_types=(
            [pltpu.VMEM((_KM,), jnp.int32)] * 8
            + [pltpu.VMEM((_KM,), jnp.float32)] * 4
            + [pltpu.VMEM((_KM,), jnp.int32)] * 4
            + [pltpu.VMEM((_KM,), jnp.int32)] * 4
            + [pltpu.VMEM((2560,), jnp.float32)]
            + [pltpu.VMEM_SHARED((_GM,), jnp.float32)]
            + [pltpu.SemaphoreType.DMA] * 20
        ),
    )(_sc_pool_body)


def _sc_pool_body(srcm_hbm, dstm_hbm, wvec_hbm, gbase_hbm, out_hbm, *scr):
    c = lax.axis_index("c")
    s = lax.axis_index("s")
    sib = scr[0:4]
    dib = scr[4:8]
    wv = scr[8:12]
    gb = scr[12:16]
    fl = scr[16:20]
    zeros_v = scr[20]
    acc = scr[21]
    ise = scr[22:26]
    dse = scr[26:30]
    wse = scr[30:34]
    gse = scr[34:38]
    sse = scr[38:42]

    def zbody(k, carry):
        zeros_v[pl.ds(16 * k, 16)] = jnp.zeros((16,), jnp.float32)
        return carry

    lax.fori_loop(0, 2560 // 16, zbody, 0)
    r0 = s * (_GM // 16)

    def zcopy(k, carry):
        pltpu.sync_copy(zeros_v, acc.at[pl.ds(r0 + k * 2560, 2560)])
        return carry

    lax.fori_loop(0, _GM // 16 // 2560, zcopy, 0)
    plsc.subcore_barrier()
    nch = _NCHM // 32
    base = (c * 16 + s) * nch

    for b in range(2):
        pltpu.async_copy(srcm_hbm.at[base + b], sib[b], ise[b])
        pltpu.async_copy(dstm_hbm.at[base + b], dib[b], dse[b])
    pltpu.make_async_copy(dstm_hbm.at[base], dib[0], dse[0]).wait()
    pltpu.async_copy(wvec_hbm.at[dib[0]], wv[0], wse[0])
    pltpu.async_copy(gbase_hbm.at[dib[0]], gb[0], gse[0])

    def step(j, b):
        si = (b + 2) % 4

        @pl.when((j >= 2) & (j + 2 < nch))
        def _():
            pltpu.make_async_copy(wv[si], acc.at[fl[si]], sse[si]).wait()

        @pl.when(j + 2 < nch)
        def _():
            pltpu.async_copy(srcm_hbm.at[base + j + 2], sib[si], ise[si])
            pltpu.async_copy(dstm_hbm.at[base + j + 2], dib[si], dse[si])

        sg = (b + 1) % 4

        @pl.when(j + 1 < nch)
        def _():
            pltpu.make_async_copy(dstm_hbm.at[base], dib[sg], dse[sg]).wait()
            pltpu.async_copy(wvec_hbm.at[dib[sg]], wv[sg], wse[sg])
            pltpu.async_copy(gbase_hbm.at[dib[sg]], gb[sg], gse[sg])

        @pl.when(j < nch)
        def _():
            pltpu.make_async_copy(srcm_hbm.at[base], sib[b], ise[b]).wait()
            pltpu.make_async_copy(wvec_hbm.at[dib[b]], wv[b], wse[b]).wait()
            pltpu.make_async_copy(gbase_hbm.at[dib[b]], gb[b], gse[b]).wait()
            for k in range(_KM // 16):
                fl[b][pl.ds(16 * k, 16)] = (gb[b][pl.ds(16 * k, 16)]
                                            + sib[b][pl.ds(16 * k, 16)])
            pltpu.async_copy(wv[b], acc.at[fl[b]], sse[b], add=True)

    def body(i, carry):
        for b in range(4):
            step(4 * i + b, b)
        return carry

    lax.fori_loop(0, (nch + 3) // 4, body, 0)
    for b in range(4):
        pltpu.make_async_copy(wv[b], acc.at[fl[b]], sse[b]).wait()
    plsc.subcore_barrier()
    pltpu.sync_copy(acc.at[pl.ds(r0, _GM // 16)],
                    out_hbm.at[c, pl.ds(r0, _GM // 16)])


def _scale_body(x_ref, dinv_ref, u_ref):
    u_ref[...] = x_ref[...] * dinv_ref[...]


_tc_scale = pl.pallas_call(
    _scale_body,
    grid=(_GRID,),
    in_specs=[
        pl.BlockSpec((_BLK, 128), lambda i: (i, 0)),
        pl.BlockSpec((_BLK, 1), lambda i: (i, 0)),
    ],
    out_specs=pl.BlockSpec((_BLK, 128), lambda i: (i, 0)),
    out_shape=jax.ShapeDtypeStruct((_NP, 128), jnp.float32),
)


def _first_body(p0_ref, p1_ref, u_ref, dinv_ref, b1_ref, w1_ref, w2_ref,
                glo_ref, ghi_ref):
    di = dinv_ref[...]
    t = (p0_ref[...] + p1_ref[...] - u_ref[...]) * di
    h1 = jnp.maximum(jnp.dot(t, w1_ref[...],
                             preferred_element_type=jnp.float32) + b1_ref[...],
                     0.0)
    g = jnp.dot(h1, w2_ref[...], preferred_element_type=jnp.float32) * di
    glo_ref[...] = g[:, :128]
    ghi_ref[...] = g[:, 128:]


_tc_first = pl.pallas_call(
    _first_body,
    grid=(_GRID,),
    in_specs=[
        pl.BlockSpec((_BLK, 128), lambda i: (i, 0)),
        pl.BlockSpec((_BLK, 128), lambda i: (i, 0)),
        pl.BlockSpec((_BLK, 128), lambda i: (i, 0)),
        pl.BlockSpec((_BLK, 1), lambda i: (i, 0)),
        pl.BlockSpec((1, 256), lambda i: (0, 0)),
        pl.BlockSpec((128, 256), lambda i: (0, 0)),
        pl.BlockSpec((256, 256), lambda i: (0, 0)),
    ],
    out_specs=[
        pl.BlockSpec((_BLK, 128), lambda i: (i, 0)),
        pl.BlockSpec((_BLK, 128), lambda i: (i, 0)),
    ],
    out_shape=[
        jax.ShapeDtypeStruct((_NP, 128), jnp.float32),
        jax.ShapeDtypeStruct((_NP, 128), jnp.float32),
    ],
)


def _layer_body(slo_ref, shi_ref, dinv_ref, b_ref, w_ref, glo_ref, ghi_ref):
    di = dinv_ref[...]
    h = jnp.concatenate([slo_ref[...], shi_ref[...]], axis=1)
    pre = jnp.maximum(h * di + b_ref[...], 0.0)
    g = jnp.dot(pre, w_ref[...], preferred_element_type=jnp.float32) * di
    glo_ref[...] = g[:, :128]
    ghi_ref[...] = g[:, 128:]


_tc_layer = pl.pallas_call(
    _layer_body,
    grid=(_GRID,),
    in_specs=[
        pl.BlockSpec((_BLK, 128), lambda i: (i, 0)),
        pl.BlockSpec((_BLK, 128), lambda i: (i, 0)),
        pl.BlockSpec((_BLK, 1), lambda i: (i, 0)),
        pl.BlockSpec((1, 256), lambda i: (0, 0)),
        pl.BlockSpec((256, 256), lambda i: (0, 0)),
    ],
    out_specs=[
        pl.BlockSpec((_BLK, 128), lambda i: (i, 0)),
        pl.BlockSpec((_BLK, 128), lambda i: (i, 0)),
    ],
    out_shape=[
        jax.ShapeDtypeStruct((_NP, 128), jnp.float32),
        jax.ShapeDtypeStruct((_NP, 128), jnp.float32),
    ],
)


def _final_body(m_ref, glo_ref, ghi_ref, b_ref, batch_ref, wl_ref, bl_ref,
                out_ref, psum, cnt):
    i = pl.program_id(0)

    @pl.when(i == 0)
    def _():
        psum[...] = jnp.zeros((_G, 256), jnp.float32)
        cnt[...] = jnp.zeros((_G, 1), jnp.float32)

    mb = m_ref[0] + m_ref[1]
    g5 = jnp.concatenate([glo_ref[...], ghi_ref[...]], axis=1)
    psum[...] += jnp.dot(mb, g5, preferred_element_type=jnp.float32)
    bt = batch_ref[0, 0, :]
    onehot = (bt[None, :] == lax.broadcasted_iota(jnp.int32, (_G, _BLK), 0)
              ).astype(jnp.float32)
    cnt[...] += jnp.sum(onehot, axis=1, keepdims=True)

    @pl.when(i == _GRID - 1)
    def _():
        pooled = (psum[...] + cnt[...] * b_ref[...]) / jnp.maximum(cnt[...],
                                                                   1.0)
        out_ref[...] = jnp.dot(pooled, wl_ref[...],
                               preferred_element_type=jnp.float32) + bl_ref[...]


_tc_final = pl.pallas_call(
    _final_body,
    grid=(_GRID,),
    in_specs=[
        pl.BlockSpec((2, _G, _BLK), lambda i: (0, 0, i)),
        pl.BlockSpec((_BLK, 128), lambda i: (i, 0)),
        pl.BlockSpec((_BLK, 128), lambda i: (i, 0)),
        pl.BlockSpec((1, 256), lambda i: (0, 0)),
        pl.BlockSpec((1, 1, _BLK), lambda i: (i, 0, 0)),
        pl.BlockSpec((256, 128), lambda i: (0, 0)),
        pl.BlockSpec((1, 128), lambda i: (0, 0)),
    ],
    out_specs=pl.BlockSpec((_G, 128), lambda i: (0, 0)),
    out_shape=jax.ShapeDtypeStruct((_G, 128), jnp.float32),
    scratch_shapes=[
        pltpu.VMEM((_G, 256), jnp.float32),
        pltpu.VMEM((_G, 1), jnp.float32),
    ],
)


def kernel(x, edge_index, batch, W1, b1, W2, b2, W3, b3, W4, b4, W5, b5,
           W_lin, b_lin):
    src2 = edge_index[0].reshape(_NCH, _K)
    dst2 = edge_index[1].reshape(_NCH, _K)
    xp = jnp.pad(x, ((0, _NP - _N), (0, 0)))
    batchp = jnp.pad(batch, (0, _NP - _N),
                     constant_values=_G).reshape(_GRID, 1, _BLK)
    loopn = jnp.arange(_NP, dtype=jnp.int32)
    srcm = jnp.concatenate([edge_index[0], loopn]).reshape(_NCHM, _KM)
    dstm = jnp.concatenate([edge_index[1], loopn]).reshape(_NCHM, _KM)

    degs = _make_sc_degree()(edge_index[1].reshape(_NCHD, _KD))
    dinv = lax.rsqrt(degs[0] + degs[1] + 1.0).reshape(_NP, 1)
    real = loopn < _N
    wvec = jnp.where(real, dinv[:, 0], 0.0)
    gbase = jnp.where(real, jnp.pad(batch, (0, _NP - _N)), 0) * _NP

    agg128 = _make_sc_agg()

    u = _tc_scale(xp, dinv)
    p0, p1 = _make_sc_agg_es()(u, src2, dst2)
    glo, ghi = _tc_first(p0, p1, u, dinv, b1.reshape(1, 256), W1, W2)
    for b_prev, W in ((b2, W3), (b3, W4), (b4, W5)):
        slo, shi = agg128(glo, ghi, src2, dst2)
        glo, ghi = _tc_layer(slo, shi, dinv, b_prev.reshape(1, 256), W)
    m = _make_sc_pool()(srcm, dstm, wvec, gbase.astype(jnp.int32))
    return _tc_final(m.reshape(2, _G, _NP), glo, ghi, b5.reshape(1, 256),
                     batchp, W_lin, b_lin.reshape(1, 128))
